# baseline (speedup 1.0000x reference)
import functools

import jax
import jax.numpy as jnp
from jax import lax
from jax.experimental import pallas as pl
from jax.experimental.pallas import tpu as pltpu
from jax.experimental.pallas import tpu_sc as plsc


_INFO = plsc.get_sparse_core_info()
_NC = _INFO.num_cores
_NS = _INFO.num_subcores
_NW = _NC * _NS


def _lane_bcast(vec, lane):
    idx = jnp.full((16, 1), lane, jnp.int32)
    dn = lax.GatherDimensionNumbers(
        offset_dims=(), collapsed_slice_dims=(0,), start_index_map=(0,)
    )
    return lax.gather(
        vec, idx, dn, slice_sizes=(1,),
        mode=lax.GatherScatterMode.PROMISE_IN_BOUNDS,
    )


def _pe_body(h, w, c, row_hbm, col_hbm, out_hbm, row_sh, col_sh, cols_v, buf_v, sem, out_sem):
    rpw = 2 * c // _NW
    row_len = h * w

    s = lax.axis_index("s")
    cc = lax.axis_index("c")
    wid = s * _NC + cc
    is_col = wid < (_NW // 2)
    cbase = jnp.where(is_col, wid, wid - _NW // 2) * rpw

    @pl.when(s == 0)
    def _stage_col():
        pltpu.sync_copy(col_hbm, col_sh)

    @pl.when(s == 1)
    def _stage_row():
        pltpu.sync_copy(row_hbm, row_sh)

    plsc.subcore_barrier()

    @pl.when(is_col)
    def _col_fetch():
        for r in range(rpw):
            pltpu.async_copy(col_sh.at[:, cbase + r], cols_v.at[r], sem)

    @pl.when(jnp.logical_not(is_col))
    def _row_fetch():
        for r in range(rpw):
            pltpu.async_copy(row_sh.at[:, cbase + r], cols_v.at[r], sem)

    for r in range(rpw):
        pltpu.make_async_copy(col_sh.at[:, 0], cols_v.at[r], sem).wait()

    half_rows = rpw // 2
    half_len = half_rows * row_len
    out_base = wid * rpw * row_len

    for piece in range(2):
        r0 = piece * half_rows

        @pl.when(is_col)
        def _col_half():
            def per_row(r, _):
                a = cols_v[r, pl.ds(0, 16)]
                b = cols_v[r, pl.ds(16, 16)]

                def per_rep(m, _):
                    base = r * row_len + m * 32
                    buf_v[pl.ds(base, 16)] = a
                    buf_v[pl.ds(base + 16, 16)] = b
                    return 0

                return lax.fori_loop(0, row_len // 32, per_rep, 0, unroll=8)

            lax.fori_loop(r0, r0 + half_rows, per_row, 0)

        @pl.when(jnp.logical_not(is_col))
        def _row_half():
            def per_row(r, _):
                a = cols_v[r, pl.ds(0, 16)]
                b = cols_v[r, pl.ds(16, 16)]

                def per_val(j, _):
                    vj = _lane_bcast(jnp.where(j < 16, a, b), j % 16)
                    base = r * row_len + j * w
                    buf_v[pl.ds(base, 16)] = vj
                    buf_v[pl.ds(base + 16, 16)] = vj
                    return 0

                return lax.fori_loop(0, h, per_val, 0, unroll=8)

            lax.fori_loop(r0, r0 + half_rows, per_row, 0)

        pltpu.async_copy(
            buf_v.at[pl.ds(piece * half_len, half_len)],
            out_hbm.at[pl.ds(out_base + piece * half_len, half_len)],
            out_sem,
        )

    for piece in range(2):
        pltpu.make_async_copy(
            buf_v.at[pl.ds(piece * half_len, half_len)],
            out_hbm.at[pl.ds(out_base + piece * half_len, half_len)],
            out_sem,
        ).wait()


def kernel(x, row_embed, col_embed):
    h, w = x.shape[-2], x.shape[-1]
    c = row_embed.shape[1]
    mesh = plsc.VectorSubcoreMesh(core_axis_name="c", subcore_axis_name="s")
    pe = pl.kernel(
        functools.partial(_pe_body, h, w, c),
        out_type=jax.ShapeDtypeStruct((2 * c * h * w,), jnp.float32),
        mesh=mesh,
        scratch_types=[
            pltpu.VMEM_SHARED((h, c), jnp.float32),
            pltpu.VMEM_SHARED((w, c), jnp.float32),
            pltpu.VMEM((2 * c // _NW, h), jnp.float32),
            pltpu.VMEM((2 * c // _NW * h * w,), jnp.float32),
            pltpu.SemaphoreType.DMA,
            pltpu.SemaphoreType.DMA,
        ],
    )
    out = pe(row_embed[:h], col_embed[:w])
    return out.reshape(2 * c, h, w)


# speedup vs baseline: 1.0263x; 1.0263x over previous
import functools

import jax
import jax.numpy as jnp
from jax import lax
from jax.experimental import pallas as pl
from jax.experimental.pallas import tpu as pltpu
from jax.experimental.pallas import tpu_sc as plsc


_INFO = plsc.get_sparse_core_info()
_NC = _INFO.num_cores
_NS = _INFO.num_subcores
_NW = _NC * _NS


def _lane_bcast(vec, lane):
    idx = jnp.full((16, 1), lane, jnp.int32)
    dn = lax.GatherDimensionNumbers(
        offset_dims=(), collapsed_slice_dims=(0,), start_index_map=(0,)
    )
    return lax.gather(
        vec, idx, dn, slice_sizes=(1,),
        mode=lax.GatherScatterMode.PROMISE_IN_BOUNDS,
    )


def _pe_body(h, w, c, row_hbm, col_hbm, out_hbm, row_sh, col_sh, cols_v, buf_v, sem):
    rpw = 2 * c // _NW
    row_len = h * w

    s = lax.axis_index("s")
    cc = lax.axis_index("c")
    wid = s * _NC + cc
    is_col = wid < (_NW // 2)
    cbase = jnp.where(is_col, wid, wid - _NW // 2) * rpw

    @pl.when(s == 0)
    def _stage_col():
        pltpu.sync_copy(col_hbm, col_sh)

    @pl.when(s == 1)
    def _stage_row():
        pltpu.sync_copy(row_hbm, row_sh)

    plsc.subcore_barrier()

    @pl.when(is_col)
    def _col_fetch():
        for r in range(rpw):
            pltpu.async_copy(col_sh.at[:, cbase + r], cols_v.at[r], sem)

    @pl.when(jnp.logical_not(is_col))
    def _row_fetch():
        for r in range(rpw):
            pltpu.async_copy(row_sh.at[:, cbase + r], cols_v.at[r], sem)

    for r in range(rpw):
        pltpu.make_async_copy(col_sh.at[:, 0], cols_v.at[r], sem).wait()

    @pl.when(is_col)
    def _col_half():
        def per_row(r, _):
            a = cols_v[r, pl.ds(0, 16)]
            b = cols_v[r, pl.ds(16, 16)]

            def per_rep(m, _):
                base = r * row_len + m * 32
                buf_v[pl.ds(base, 16)] = a
                buf_v[pl.ds(base + 16, 16)] = b
                return 0

            return lax.fori_loop(0, row_len // 32, per_rep, 0, unroll=8)

        lax.fori_loop(0, rpw, per_row, 0)

    @pl.when(jnp.logical_not(is_col))
    def _row_half():
        def per_row(r, _):
            a = cols_v[r, pl.ds(0, 16)]
            b = cols_v[r, pl.ds(16, 16)]

            def per_val(j, _):
                vj = _lane_bcast(jnp.where(j < 16, a, b), j % 16)
                base = r * row_len + j * w
                buf_v[pl.ds(base, 16)] = vj
                buf_v[pl.ds(base + 16, 16)] = vj
                return 0

            return lax.fori_loop(0, h, per_val, 0, unroll=8)

        lax.fori_loop(0, rpw, per_row, 0)

    pltpu.sync_copy(buf_v, out_hbm.at[pl.ds(wid * rpw * row_len, rpw * row_len)])


def kernel(x, row_embed, col_embed):
    h, w = x.shape[-2], x.shape[-1]
    c = row_embed.shape[1]
    mesh = plsc.VectorSubcoreMesh(core_axis_name="c", subcore_axis_name="s")
    pe = pl.kernel(
        functools.partial(_pe_body, h, w, c),
        out_type=jax.ShapeDtypeStruct((2 * c * h * w,), jnp.float32),
        mesh=mesh,
        scratch_types=[
            pltpu.VMEM_SHARED((h, c), jnp.float32),
            pltpu.VMEM_SHARED((w, c), jnp.float32),
            pltpu.VMEM((2 * c // _NW, h), jnp.float32),
            pltpu.VMEM((2 * c // _NW * h * w,), jnp.float32),
            pltpu.SemaphoreType.DMA,
        ],
    )
    out = pe(row_embed[:h], col_embed[:w])
    return out.reshape(2 * c, h, w)


# speedup vs baseline: 1.0414x; 1.0147x over previous
import functools

import jax
import jax.numpy as jnp
from jax import lax
from jax.experimental import pallas as pl
from jax.experimental.pallas import tpu as pltpu
from jax.experimental.pallas import tpu_sc as plsc


_INFO = plsc.get_sparse_core_info()
_NC = 1
_NS = _INFO.num_subcores
_NW = _NC * _NS


def _lane_bcast(vec, lane):
    idx = jnp.full((16, 1), lane, jnp.int32)
    dn = lax.GatherDimensionNumbers(
        offset_dims=(), collapsed_slice_dims=(0,), start_index_map=(0,)
    )
    return lax.gather(
        vec, idx, dn, slice_sizes=(1,),
        mode=lax.GatherScatterMode.PROMISE_IN_BOUNDS,
    )


def _pe_body(h, w, c, row_hbm, col_hbm, out_hbm, row_sh, col_sh, cols_v, buf_v, sem):
    rpw = 2 * c // _NW
    row_len = h * w

    s = lax.axis_index("s")
    cc = lax.axis_index("c")
    wid = s * _NC + cc
    is_col = wid < (_NW // 2)
    cbase = jnp.where(is_col, wid, wid - _NW // 2) * rpw

    @pl.when(s == 0)
    def _stage_col():
        pltpu.sync_copy(col_hbm, col_sh)

    @pl.when(s == 1)
    def _stage_row():
        pltpu.sync_copy(row_hbm, row_sh)

    plsc.subcore_barrier()

    @pl.when(is_col)
    def _col_fetch():
        for r in range(rpw):
            pltpu.async_copy(col_sh.at[:, cbase + r], cols_v.at[r], sem)

    @pl.when(jnp.logical_not(is_col))
    def _row_fetch():
        for r in range(rpw):
            pltpu.async_copy(row_sh.at[:, cbase + r], cols_v.at[r], sem)

    for r in range(rpw):
        pltpu.make_async_copy(col_sh.at[:, 0], cols_v.at[r], sem).wait()

    @pl.when(is_col)
    def _col_half():
        def per_row(r, _):
            a = cols_v[r, pl.ds(0, 16)]
            b = cols_v[r, pl.ds(16, 16)]

            def per_rep(m, _):
                base = r * row_len + m * 32
                buf_v[pl.ds(base, 16)] = a
                buf_v[pl.ds(base + 16, 16)] = b
                return 0

            return lax.fori_loop(0, row_len // 32, per_rep, 0, unroll=8)

        lax.fori_loop(0, rpw, per_row, 0)

    @pl.when(jnp.logical_not(is_col))
    def _row_half():
        def per_row(r, _):
            a = cols_v[r, pl.ds(0, 16)]
            b = cols_v[r, pl.ds(16, 16)]

            def per_val(j, _):
                vj = _lane_bcast(jnp.where(j < 16, a, b), j % 16)
                base = r * row_len + j * w
                buf_v[pl.ds(base, 16)] = vj
                buf_v[pl.ds(base + 16, 16)] = vj
                return 0

            return lax.fori_loop(0, h, per_val, 0, unroll=8)

        lax.fori_loop(0, rpw, per_row, 0)

    pltpu.sync_copy(buf_v, out_hbm.at[pl.ds(wid * rpw * row_len, rpw * row_len)])


def kernel(x, row_embed, col_embed):
    h, w = x.shape[-2], x.shape[-1]
    c = row_embed.shape[1]
    mesh = plsc.VectorSubcoreMesh(core_axis_name="c", subcore_axis_name="s", num_cores=1)
    pe = pl.kernel(
        functools.partial(_pe_body, h, w, c),
        out_type=jax.ShapeDtypeStruct((2 * c * h * w,), jnp.float32),
        mesh=mesh,
        scratch_types=[
            pltpu.VMEM_SHARED((h, c), jnp.float32),
            pltpu.VMEM_SHARED((w, c), jnp.float32),
            pltpu.VMEM((2 * c // _NW, h), jnp.float32),
            pltpu.VMEM((2 * c // _NW * h * w,), jnp.float32),
            pltpu.SemaphoreType.DMA,
        ],
    )
    out = pe(row_embed[:h], col_embed[:w])
    return out.reshape(2 * c, h, w)
